# Initial kernel scaffold; baseline (speedup 1.0000x reference)
import jax
import jax.numpy as jnp
from jax.experimental import pallas as pl


def kernel(tgt_tok_id):
    raise NotImplementedError("write your pallas kernel here")


# speedup vs baseline: 1.3797x; 1.3797x over previous
import jax
import jax.numpy as jnp
from jax.experimental import pallas as pl

_SMOOTHING = 0.1
_CONFIDENCE = 1.0 - _SMOOTHING
_V = 100000
_B = 1024
_FILL = _SMOOTHING / (_V - 2)

_VB = 2048


def _smooth_block(ids_ref, out_ref):
    j = pl.program_id(0)
    ids = ids_ref[0, :]
    cols = jax.lax.broadcasted_iota(jnp.int32, (_B, _VB), 1) + j * _VB
    val = jnp.where(cols == ids[:, None], _CONFIDENCE, _FILL)
    out_ref[...] = jnp.where(cols == 0, 0.0, val)


def kernel(tgt_tok_id):
    ids = tgt_tok_id.reshape(1, _B).astype(jnp.int32)
    n_blocks = pl.cdiv(_V, _VB)
    return pl.pallas_call(
        _smooth_block,
        grid=(n_blocks,),
        in_specs=[pl.BlockSpec((1, _B), lambda j: (0, 0))],
        out_specs=pl.BlockSpec((_B, _VB), lambda j: (0, j)),
        out_shape=jax.ShapeDtypeStruct((_B, _V), jnp.float32),
    )(ids)
